# baseline (speedup 1.0000x reference)
import functools

import jax
import jax.numpy as jnp
from jax import lax
from jax.experimental import pallas as pl
from jax.experimental.pallas import tpu as pltpu
from jax.experimental.pallas import tpu_sc as plsc

N = 100000
D = 128
H = 128
B = 512
EPS = 1e-5

R = 2000
NBLK = N // R

_NC = 2
_NS = 16
_NW = _NC * _NS
_BPW = B // _NW


def _gather_last_rows(feat, last_nodes):
    mesh = plsc.VectorSubcoreMesh(core_axis_name="c", subcore_axis_name="s")

    @functools.partial(
        pl.kernel,
        mesh=mesh,
        out_type=jax.ShapeDtypeStruct((B, D), jnp.float32),
        scratch_types=[
            pltpu.VMEM((_BPW,), jnp.int32),
            pltpu.VMEM((_BPW, D), jnp.float32),
            pltpu.SemaphoreType.DMA,
        ],
    )
    def k(table_hbm, idx_hbm, out_hbm, idx_v, rows_v, sem):
        wid = lax.axis_index("s") * _NC + lax.axis_index("c")
        base = wid * _BPW
        pltpu.sync_copy(idx_hbm.at[pl.ds(base, _BPW)], idx_v)
        pltpu.async_copy(table_hbm.at[idx_v], rows_v, sem).wait()
        pltpu.sync_copy(rows_v, out_hbm.at[pl.ds(base, _BPW)])

    return k(feat, last_nodes)


_NT = (((1,), (1,)), ((), ()))
_TN = (((0,), (0,)), ((), ()))


def _fused_body(x_ref, seg_ref, gamma_ref, beta_ref,
                wu_ref, gath_ref, wv_ref, bv_ref, we_ref, out_ref,
                st_ref, ab_ref, d_ref, s_ref, fv_ref):
    i = pl.program_id(0)

    @pl.when(i == 0)
    def _zero():
        st_ref[...] = jnp.zeros((2, D), jnp.float32)

    @pl.when(i < NBLK)
    def _stats():
        x = x_ref[...]
        st_ref[0:1, :] = st_ref[0:1, :] + jnp.sum(x, axis=0, keepdims=True)
        st_ref[1:2, :] = st_ref[1:2, :] + jnp.sum(x * x, axis=0, keepdims=True)

    @pl.when(i == NBLK)
    def _init():
        mean = st_ref[0:1, :] * (1.0 / N)
        var = st_ref[1:2, :] * (1.0 / N) - mean * mean
        a = gamma_ref[...] * lax.rsqrt(var + EPS)
        ab_ref[0:1, :] = a
        ab_ref[1:2, :] = beta_ref[...] - mean * a
        d_ref[...] = jnp.zeros((B, 1), jnp.float32)
        s_ref[...] = jnp.zeros((B, D), jnp.float32)
        fv_ref[...] = (
            lax.dot_general(gath_ref[...] * a + ab_ref[1:2, :], wv_ref[...],
                            _NT, preferred_element_type=jnp.float32)
            + bv_ref[...]
        )

    @pl.when(i >= NBLK)
    def _main():
        a = ab_ref[0:1, :]
        b = ab_ref[1:2, :]
        f = x_ref[...] * a + b
        u = lax.dot_general(f, wu_ref[...], _NT,
                            preferred_element_type=jnp.float32)

        segr = seg_ref[0]
        ohT = segr == lax.broadcasted_iota(jnp.int32, (B, R), 0)
        ohT32 = ohT.astype(jnp.float32)

        vb = lax.dot_general(ohT32, fv_ref[...], _TN,
                             preferred_element_type=jnp.float32)
        sg = jax.nn.sigmoid(u + vb)
        e = lax.dot_general(we_ref[...], sg, _NT,
                            preferred_element_type=jnp.float32)

        C = jnp.sum(jnp.abs(we_ref[...]))
        w = jnp.exp(e - C)
        ohTw = ohT32 * w
        d_ref[...] = d_ref[...] + jnp.sum(ohTw, axis=1, keepdims=True)
        s_ref[...] = s_ref[...] + jnp.dot(ohTw, f,
                                          preferred_element_type=jnp.float32)

        @pl.when(i == 2 * NBLK - 1)
        def _fin():
            d_c = d_ref[...]
            out_ref[...] = jnp.where(d_c > 0.0, s_ref[...] / d_c, 0.0)


def _fused_pass(feat, seg3d, gamma2, beta2, wu, gathered, wv, bv2, we2):
    return pl.pallas_call(
        _fused_body,
        grid=(2 * NBLK,),
        in_specs=[
            pl.BlockSpec((R, D), lambda i: (i % NBLK, 0)),
            pl.BlockSpec((1, 1, R), lambda i: (i % NBLK, 0, 0)),
            pl.BlockSpec((1, D), lambda i: (0, 0)),
            pl.BlockSpec((1, D), lambda i: (0, 0)),
            pl.BlockSpec((D, H), lambda i: (0, 0)),
            pl.BlockSpec((B, D), lambda i: (0, 0)),
            pl.BlockSpec((D, H), lambda i: (0, 0)),
            pl.BlockSpec((1, H), lambda i: (0, 0)),
            pl.BlockSpec((1, H), lambda i: (0, 0)),
        ],
        out_specs=pl.BlockSpec((B, D), lambda i: (0, 0)),
        out_shape=jax.ShapeDtypeStruct((B, D), jnp.float32),
        scratch_shapes=[
            pltpu.VMEM((2, D), jnp.float32),
            pltpu.VMEM((2, D), jnp.float32),
            pltpu.VMEM((B, 1), jnp.float32),
            pltpu.VMEM((B, D), jnp.float32),
            pltpu.VMEM((B, H), jnp.float32),
        ],
    )(feat, seg3d, gamma2, beta2, wu, gathered, wv, bv2, we2)


def kernel(feat, segment_ids, last_nodes, gamma, beta, W_u, W_v, b_v, W_e):
    gathered = _gather_last_rows(feat, last_nodes)
    seg3d = segment_ids.reshape(NBLK, 1, R)
    return _fused_pass(
        feat, seg3d,
        gamma.reshape(1, D), beta.reshape(1, D),
        W_u, gathered, W_v, b_v.reshape(1, H), W_e.reshape(1, H),
    )


# speedup vs baseline: 1.1195x; 1.1195x over previous
import functools

import jax
import jax.numpy as jnp
from jax import lax
from jax.experimental import pallas as pl
from jax.experimental.pallas import tpu as pltpu
from jax.experimental.pallas import tpu_sc as plsc

N = 100000
D = 128
H = 128
B = 512
EPS = 1e-5

R = 4000
NBLK = N // R

_NC = 2
_NS = 16
_NW = _NC * _NS
_BPW = B // _NW


def _gather_last_rows(feat, last_nodes):
    mesh = plsc.VectorSubcoreMesh(core_axis_name="c", subcore_axis_name="s")

    @functools.partial(
        pl.kernel,
        mesh=mesh,
        out_type=jax.ShapeDtypeStruct((B, D), jnp.float32),
        scratch_types=[
            pltpu.VMEM((_BPW,), jnp.int32),
            pltpu.VMEM((_BPW, D), jnp.float32),
            pltpu.SemaphoreType.DMA,
        ],
    )
    def k(table_hbm, idx_hbm, out_hbm, idx_v, rows_v, sem):
        wid = lax.axis_index("s") * _NC + lax.axis_index("c")
        base = wid * _BPW
        pltpu.sync_copy(idx_hbm.at[pl.ds(base, _BPW)], idx_v)
        pltpu.async_copy(table_hbm.at[idx_v], rows_v, sem).wait()
        pltpu.sync_copy(rows_v, out_hbm.at[pl.ds(base, _BPW)])

    return k(feat, last_nodes)


def _stats_body(x_ref, out_ref):
    i = pl.program_id(0)

    @pl.when(i == 0)
    def _():
        out_ref[...] = jnp.zeros_like(out_ref)

    x = x_ref[...]
    out_ref[0:1, :] = out_ref[0:1, :] + jnp.sum(x, axis=0, keepdims=True)
    out_ref[1:2, :] = out_ref[1:2, :] + jnp.sum(x * x, axis=0, keepdims=True)


def _col_stats(feat):
    return pl.pallas_call(
        _stats_body,
        grid=(NBLK,),
        in_specs=[pl.BlockSpec((R, D), lambda i: (i, 0))],
        out_specs=pl.BlockSpec((2, D), lambda i: (0, 0)),
        out_shape=jax.ShapeDtypeStruct((2, D), jnp.float32),
    )(feat)


_NT = (((1,), (1,)), ((), ()))
_TN = (((0,), (0,)), ((), ()))


def _main_body(x_ref, seg_ref, stats_ref, gamma_ref, beta_ref,
               wu_ref, gath_ref, wv_ref, bv_ref, we_ref, out_ref,
               d_ref, s_ref, fv_ref):
    i = pl.program_id(0)

    mean = stats_ref[0:1, :] * (1.0 / N)
    var = stats_ref[1:2, :] * (1.0 / N) - mean * mean
    a = gamma_ref[...] * lax.rsqrt(var + EPS)
    b = beta_ref[...] - mean * a

    @pl.when(i == 0)
    def _init():
        d_ref[...] = jnp.zeros((B, 1), jnp.float32)
        s_ref[...] = jnp.zeros((B, D), jnp.float32)
        fv_ref[...] = (
            lax.dot_general(gath_ref[...] * a + b, wv_ref[...], _NT,
                            preferred_element_type=jnp.float32)
            + bv_ref[...]
        )

    f = x_ref[...] * a + b
    u = lax.dot_general(f, wu_ref[...], _NT,
                        preferred_element_type=jnp.float32)

    segr = seg_ref[0]
    ohT = segr == lax.broadcasted_iota(jnp.int32, (B, R), 0)
    ohT32 = ohT.astype(jnp.float32)

    vb = lax.dot_general(ohT32, fv_ref[...], _TN,
                         preferred_element_type=jnp.float32)
    sg = jax.nn.sigmoid(u + vb)
    e = lax.dot_general(we_ref[...], sg, _NT,
                        preferred_element_type=jnp.float32)

    C = jnp.sum(jnp.abs(we_ref[...]))
    w = jnp.exp(e - C)
    ohTw = ohT32 * w
    d_ref[...] = d_ref[...] + jnp.sum(ohTw, axis=1, keepdims=True)
    s_ref[...] = s_ref[...] + jnp.dot(ohTw, f,
                                      preferred_element_type=jnp.float32)

    @pl.when(i == NBLK - 1)
    def _fin():
        d_c = d_ref[...]
        out_ref[...] = jnp.where(d_c > 0.0, s_ref[...] / d_c, 0.0)


def _main_pass(feat, seg3d, stats, gamma2, beta2, wut, gathered,
               wvt, bv2, we2):
    return pl.pallas_call(
        _main_body,
        grid=(NBLK,),
        in_specs=[
            pl.BlockSpec((R, D), lambda i: (i, 0)),
            pl.BlockSpec((1, 1, R), lambda i: (i, 0, 0)),
            pl.BlockSpec((2, D), lambda i: (0, 0)),
            pl.BlockSpec((1, D), lambda i: (0, 0)),
            pl.BlockSpec((1, D), lambda i: (0, 0)),
            pl.BlockSpec((D, H), lambda i: (0, 0)),
            pl.BlockSpec((B, D), lambda i: (0, 0)),
            pl.BlockSpec((D, H), lambda i: (0, 0)),
            pl.BlockSpec((1, H), lambda i: (0, 0)),
            pl.BlockSpec((1, H), lambda i: (0, 0)),
        ],
        out_specs=pl.BlockSpec((B, D), lambda i: (0, 0)),
        out_shape=jax.ShapeDtypeStruct((B, D), jnp.float32),
        scratch_shapes=[
            pltpu.VMEM((B, 1), jnp.float32),
            pltpu.VMEM((B, D), jnp.float32),
            pltpu.VMEM((B, H), jnp.float32),
        ],
    )(feat, seg3d, stats, gamma2, beta2, wut, gathered, wvt, bv2, we2)


def kernel(feat, segment_ids, last_nodes, gamma, beta, W_u, W_v, b_v, W_e):
    gathered = _gather_last_rows(feat, last_nodes)
    stats = _col_stats(feat)

    seg3d = segment_ids.reshape(NBLK, 1, R)
    return _main_pass(
        feat, seg3d, stats,
        gamma.reshape(1, D), beta.reshape(1, D),
        W_u, gathered, W_v, b_v.reshape(1, H), W_e.reshape(1, H),
    )
